# baseline (speedup 1.0000x reference)
import functools

import jax
import jax.numpy as jnp
from jax import lax
from jax.experimental import pallas as pl
from jax.experimental.pallas import tpu as pltpu
from jax.experimental.pallas import tpu_sc as plsc

NUM_SEG = 16
TOTAL_TOK = 32768
D = 256
L = 16

NC = 2
NS = 16
COLS = D // NC
CW = COLS // L

SC_ROWS = 8192
TOK_PER_TILE = SC_ROWS // NS
CHUNK = 128
NCHUNK = TOK_PER_TILE // CHUNK
GROUPS = CHUNK // L
NBUF = 4


def _sc_segment_sum(data, ids):
  mesh = plsc.VectorSubcoreMesh(core_axis_name="c", subcore_axis_name="s")

  @functools.partial(
      pl.kernel,
      out_type=jax.ShapeDtypeStruct((NUM_SEG, D), jnp.float32),
      mesh=mesh,
      scratch_types=[
          pltpu.VMEM((NBUF, CHUNK, COLS), jnp.float32),
          pltpu.VMEM((NBUF, CHUNK), jnp.int32),
          pltpu.VMEM((NUM_SEG, COLS), jnp.float32),
          pltpu.VMEM((NS, COLS), jnp.float32),
          pltpu.VMEM((COLS,), jnp.float32),
          pltpu.VMEM_SHARED((NS, NUM_SEG, COLS), jnp.float32),
          pltpu.SemaphoreType.DMA((NBUF,)),
      ],
  )
  def k(data_hbm, ids_hbm, out_hbm, dbuf, ibuf, acc, tbuf, obuf, shared,
        sems):
    cid = lax.axis_index("c")
    sid = lax.axis_index("s")
    row0 = sid * TOK_PER_TILE

    zero = jnp.zeros((L,), jnp.float32)
    for s in range(NUM_SEG):
      for d in range(CW):
        acc[s, pl.ds(d * L, L)] = zero

    col0 = cid * COLS

    def start(ch, b):
      base = row0 + ch * CHUNK
      pltpu.async_copy(
          data_hbm.at[pl.ds(base, CHUNK), pl.ds(col0, COLS)], dbuf.at[b],
          sems.at[b]
      )
      pltpu.async_copy(ids_hbm.at[pl.ds(base, CHUNK)], ibuf.at[b],
                       sems.at[b])

    def drain(b):
      pltpu.make_async_copy(
          data_hbm.at[pl.ds(0, CHUNK), pl.ds(col0, COLS)], dbuf.at[b],
          sems.at[b]
      ).wait()
      pltpu.make_async_copy(
          ids_hbm.at[pl.ds(0, CHUNK)], ibuf.at[b], sems.at[b]
      ).wait()

    def compute(b):
      def group_body(g):
        segs = ibuf[b, pl.ds(g * L, L)]
        s_first = segs[0]
        s_last = segs[L - 1]

        @pl.when(s_first == s_last)
        def _uniform():
          for d in range(CW):
            sl = pl.ds(d * L, L)
            vs = [dbuf[b, g * L + j, sl] for j in range(L)]
            while len(vs) > 1:
              vs = [vs[i] + vs[i + 1] for i in range(0, len(vs), 2)]
            plsc.addupdate(acc.at[s_first, sl], vs[0])

        @pl.when(s_first != s_last)
        def _mixed():
          for j in range(L):
            seg = segs[j]
            for d in range(CW):
              sl = pl.ds(d * L, L)
              plsc.addupdate(acc.at[seg, sl], dbuf[b, g * L + j, sl])

      pl.loop(0, GROUPS, unroll=2)(group_body)

    for b in range(min(NBUF, NCHUNK)):
      start(b, b)

    for ch in range(NCHUNK):
      b = ch % NBUF
      drain(b)
      compute(b)
      if ch + NBUF < NCHUNK:
        start(ch + NBUF, b)

    pltpu.sync_copy(acc, shared.at[sid])
    plsc.subcore_barrier()
    pltpu.sync_copy(shared.at[:, sid], tbuf)
    for d in range(CW):
      sl = pl.ds(d * L, L)
      v = tbuf[0, sl]
      for t in range(1, NS):
        v = v + tbuf[t, sl]
      obuf[sl] = v
    pltpu.sync_copy(obuf, out_hbm.at[sid, pl.ds(col0, COLS)])

  return k(data, ids)


TC_BLK = 4096


def _tc_body(ids_ref, data_ref, out_ref):
  i = pl.program_id(0)
  oh_t = (ids_ref[...] == lax.broadcasted_iota(
      jnp.int32, (NUM_SEG, TC_BLK), 0)).astype(jnp.float32)
  contrib = lax.dot_general(
      oh_t, data_ref[...], (((1,), (0,)), ((), ())),
      preferred_element_type=jnp.float32)

  @pl.when(i == 0)
  def _init():
    out_ref[...] = jnp.zeros_like(out_ref)

  out_ref[...] += contrib


def _tc_segment_sum(data, ids_row, row_start, row_end):
  nblk = (row_end - row_start) // TC_BLK
  blk0 = row_start // TC_BLK
  return pl.pallas_call(
      _tc_body,
      grid=(nblk,),
      in_specs=[
          pl.BlockSpec((1, TC_BLK), lambda i: (0, blk0 + i)),
          pl.BlockSpec((TC_BLK, D), lambda i: (blk0 + i, 0)),
      ],
      out_specs=pl.BlockSpec((NUM_SEG, D), lambda i: (0, 0)),
      out_shape=jax.ShapeDtypeStruct((NUM_SEG, D), jnp.float32),
  )(ids_row, data)


@jax.jit
def kernel(data, segment_ids):
  ids = segment_ids.astype(jnp.int32)
  sc_part = _sc_segment_sum(data, ids)
  tc_part = _tc_segment_sum(data, ids[None, :], SC_ROWS, TOTAL_TOK)
  return sc_part + tc_part


# speedup vs baseline: 1.2986x; 1.2986x over previous
import functools

import jax
import jax.numpy as jnp
from jax import lax
from jax.experimental import pallas as pl
from jax.experimental.pallas import tpu as pltpu
from jax.experimental.pallas import tpu_sc as plsc

NUM_SEG = 16
TOTAL_TOK = 32768
D = 256
L = 16

NC = 2
NS = 16
COLS = D // NC
CW = COLS // L

SC_ROWS = 4096
TOK_PER_TILE = SC_ROWS // NS
CHUNK = 256
NCHUNK = TOK_PER_TILE // CHUNK
GROUPS = CHUNK // L
NBUF = 2


def _sc_segment_sum(data, ids):
  mesh = plsc.VectorSubcoreMesh(core_axis_name="c", subcore_axis_name="s")

  @functools.partial(
      pl.kernel,
      out_type=jax.ShapeDtypeStruct((NUM_SEG, D), jnp.float32),
      mesh=mesh,
      scratch_types=[
          pltpu.VMEM((NBUF, CHUNK, COLS), jnp.float32),
          pltpu.VMEM((NBUF, CHUNK), jnp.int32),
          pltpu.VMEM((NUM_SEG, COLS), jnp.float32),
          pltpu.VMEM((NS, COLS), jnp.float32),
          pltpu.VMEM((COLS,), jnp.float32),
          pltpu.VMEM_SHARED((NS, NUM_SEG, COLS), jnp.float32),
          pltpu.SemaphoreType.DMA((NBUF,)),
      ],
  )
  def k(data_hbm, ids_hbm, out_hbm, dbuf, ibuf, acc, tbuf, obuf, shared,
        sems):
    cid = lax.axis_index("c")
    sid = lax.axis_index("s")
    row0 = sid * TOK_PER_TILE

    zero = jnp.zeros((L,), jnp.float32)
    for s in range(NUM_SEG):
      for d in range(CW):
        acc[s, pl.ds(d * L, L)] = zero

    col0 = cid * COLS

    def start(ch, b):
      base = row0 + ch * CHUNK
      pltpu.async_copy(
          data_hbm.at[pl.ds(base, CHUNK), pl.ds(col0, COLS)], dbuf.at[b],
          sems.at[b]
      )
      pltpu.async_copy(ids_hbm.at[pl.ds(base, CHUNK)], ibuf.at[b],
                       sems.at[b])

    def drain(b):
      pltpu.make_async_copy(
          data_hbm.at[pl.ds(0, CHUNK), pl.ds(col0, COLS)], dbuf.at[b],
          sems.at[b]
      ).wait()
      pltpu.make_async_copy(
          ids_hbm.at[pl.ds(0, CHUNK)], ibuf.at[b], sems.at[b]
      ).wait()

    def compute(b):
      def group_body(g):
        segs = ibuf[b, pl.ds(g * L, L)]
        s_first = segs[0]
        s_last = segs[L - 1]

        @pl.when(s_first == s_last)
        def _uniform():
          for d in range(CW):
            sl = pl.ds(d * L, L)
            vs = [dbuf[b, g * L + j, sl] for j in range(L)]
            while len(vs) > 1:
              vs = [vs[i] + vs[i + 1] for i in range(0, len(vs), 2)]
            plsc.addupdate(acc.at[s_first, sl], vs[0])

        @pl.when(s_first != s_last)
        def _mixed():
          for j in range(L):
            seg = segs[j]
            for d in range(CW):
              sl = pl.ds(d * L, L)
              plsc.addupdate(acc.at[seg, sl], dbuf[b, g * L + j, sl])

      pl.loop(0, GROUPS, unroll=2)(group_body)

    for b in range(min(NBUF, NCHUNK)):
      start(b, b)

    for ch in range(NCHUNK):
      b = ch % NBUF
      drain(b)
      compute(b)
      if ch + NBUF < NCHUNK:
        start(ch + NBUF, b)

    pltpu.sync_copy(acc, shared.at[sid])
    plsc.subcore_barrier()
    pltpu.sync_copy(shared.at[:, sid], tbuf)
    for d in range(CW):
      sl = pl.ds(d * L, L)
      v = tbuf[0, sl]
      for t in range(1, NS):
        v = v + tbuf[t, sl]
      obuf[sl] = v
    pltpu.sync_copy(obuf, out_hbm.at[sid, pl.ds(col0, COLS)])

  return k(data, ids)


TC_BLK = 4096


def _tc_body(ids_ref, data_ref, out_ref):
  i = pl.program_id(0)
  oh_t = (ids_ref[...] == lax.broadcasted_iota(
      jnp.int32, (NUM_SEG, TC_BLK), 0)).astype(jnp.float32)
  contrib = lax.dot_general(
      oh_t, data_ref[...], (((1,), (0,)), ((), ())),
      preferred_element_type=jnp.float32)

  @pl.when(i == 0)
  def _init():
    out_ref[...] = jnp.zeros_like(out_ref)

  out_ref[...] += contrib


def _tc_segment_sum(data, ids_row, row_start, row_end):
  nblk = (row_end - row_start) // TC_BLK
  blk0 = row_start // TC_BLK
  return pl.pallas_call(
      _tc_body,
      grid=(nblk,),
      in_specs=[
          pl.BlockSpec((1, TC_BLK), lambda i: (0, blk0 + i)),
          pl.BlockSpec((TC_BLK, D), lambda i: (blk0 + i, 0)),
      ],
      out_specs=pl.BlockSpec((NUM_SEG, D), lambda i: (0, 0)),
      out_shape=jax.ShapeDtypeStruct((NUM_SEG, D), jnp.float32),
  )(ids_row, data)


@jax.jit
def kernel(data, segment_ids):
  ids = segment_ids.astype(jnp.int32)
  sc_part = _sc_segment_sum(data, ids)
  tc_part = _tc_segment_sum(data, ids[None, :], SC_ROWS, TOTAL_TOK)
  return sc_part + tc_part
